# baseline (speedup 1.0000x reference)
import functools

import jax
import jax.numpy as jnp
from jax import lax
from jax.experimental import pallas as pl
from jax.experimental.pallas import tpu as pltpu
from jax.experimental.pallas import tpu_sc as plsc

_N = 16384
_D = 2048
_NW = 32
_PIECE = 16
_NBUF = 3
_LA = 2


def _sc_copy(x_hbm, y_hbm, out_hbm, bufs, in_sems, out_sems):
    wid = lax.axis_index("s") * 2 + lax.axis_index("c")
    rows_per_w = _N // _NW
    base = wid * rows_per_w
    n_pieces = rows_per_w // _PIECE

    def in_cp(p, slot):
        return pltpu.make_async_copy(
            y_hbm.at[pl.ds(base + p * _PIECE, _PIECE), :], bufs.at[slot],
            in_sems.at[slot])

    def out_cp(p, slot):
        return pltpu.make_async_copy(
            bufs.at[slot], out_hbm.at[pl.ds(base + p * _PIECE, _PIECE), :],
            out_sems.at[slot])

    for i in range(_LA):
        in_cp(i, i).start()

    def body(p, _):
        slot = lax.rem(p, _NBUF)
        in_cp(p, slot).wait()
        out_cp(p, slot).start()

        j = p + _LA
        jslot = lax.rem(j, _NBUF)

        @pl.when(j < n_pieces)
        def _prefetch():
            @pl.when(j >= _NBUF)
            def _slot_free():
                out_cp(j - _NBUF, jslot).wait()
            in_cp(j, jslot).start()
        return _

    lax.fori_loop(0, n_pieces, body, 0)

    def drain(p, _):
        @pl.when(p >= n_pieces - _NBUF)
        def _w():
            out_cp(p, lax.rem(p, _NBUF)).wait()
        return _
    lax.fori_loop(n_pieces - _NBUF, n_pieces, drain, 0)


def kernel(x, y):
    n, d = y.shape
    mesh = plsc.VectorSubcoreMesh(core_axis_name="c", subcore_axis_name="s")
    k = functools.partial(
        pl.kernel,
        out_type=jax.ShapeDtypeStruct((n, d), y.dtype),
        mesh=mesh,
        scratch_types=[
            pltpu.VMEM((_NBUF, _PIECE, _D), jnp.float32),
            pltpu.SemaphoreType.DMA((_NBUF,)),
            pltpu.SemaphoreType.DMA((_NBUF,)),
        ],
    )(_sc_copy)
    return k(x.astype(jnp.int32), y)


# speedup vs baseline: 1.3438x; 1.3438x over previous
import jax
import jax.numpy as jnp
from jax.experimental import pallas as pl
from jax.experimental.pallas import tpu as pltpu

_K = 8
_R = 1024


def _body(x_vmem, x_smem, y_ref, out_ref):
    b = pl.program_id(0)
    sub = _R // 128
    labels = x_vmem[pl.ds(b * sub, sub), :]
    n_bad = jnp.sum(((labels < 0) | (labels >= _K)).astype(jnp.int32))

    out_ref[:, :] = y_ref[:, :]

    @pl.when(n_bad > 0)
    def _fixup():
        def zero_bad_row(i, _):
            lab = x_smem[b * _R + i]

            @pl.when((lab < 0) | (lab >= _K))
            def _z():
                out_ref[pl.ds(i, 1), :] = jnp.zeros((1, out_ref.shape[1]),
                                                    out_ref.dtype)
            return _
        jax.lax.fori_loop(0, _R, zero_bad_row, 0)


def kernel(x, y):
    n, d = y.shape
    grid = n // _R
    xi = x.astype(jnp.int32)
    x2 = xi.reshape(n // 128, 128)
    return pl.pallas_call(
        _body,
        grid=(grid,),
        in_specs=[
            pl.BlockSpec((n // 128, 128), lambda i: (0, 0)),
            pl.BlockSpec(memory_space=pltpu.MemorySpace.SMEM),
            pl.BlockSpec((_R, d), lambda i: (i, 0)),
        ],
        out_specs=pl.BlockSpec((_R, d), lambda i: (i, 0)),
        out_shape=jax.ShapeDtypeStruct((n, d), y.dtype),
        compiler_params=pltpu.CompilerParams(
            dimension_semantics=("arbitrary",),
        ),
    )(x2, xi, y)


# speedup vs baseline: 1.3449x; 1.0008x over previous
import jax
import jax.numpy as jnp
from jax.experimental import pallas as pl
from jax.experimental.pallas import tpu as pltpu

_K = 8
_R = 1024


def _body(x_vmem, x_smem, y_ref, out_ref):
    b = pl.program_id(0)
    sub = _R // 128
    labels = x_vmem[pl.ds(b * sub, sub), :]
    n_bad = jnp.sum(((labels < 0) | (labels >= _K)).astype(jnp.int32))

    out_ref[:, :] = y_ref[:, :]

    @pl.when(n_bad > 0)
    def _fixup():
        def zero_bad_row(i, _):
            lab = x_smem[b * _R + i]

            @pl.when((lab < 0) | (lab >= _K))
            def _z():
                out_ref[pl.ds(i, 1), :] = jnp.zeros((1, out_ref.shape[1]),
                                                    out_ref.dtype)
            return _
        jax.lax.fori_loop(0, _R, zero_bad_row, 0)


def kernel(x, y):
    n, d = y.shape
    grid = n // _R
    xi = x.astype(jnp.int32)
    x2 = xi.reshape(n // 128, 128)
    return pl.pallas_call(
        _body,
        grid=(grid,),
        in_specs=[
            pl.BlockSpec((n // 128, 128), lambda i: (0, 0)),
            pl.BlockSpec(memory_space=pltpu.MemorySpace.SMEM),
            pl.BlockSpec((_R, d), lambda i: (i, 0)),
        ],
        out_specs=pl.BlockSpec((_R, d), lambda i: (i, 0)),
        out_shape=jax.ShapeDtypeStruct((n, d), y.dtype),
        compiler_params=pltpu.CompilerParams(
            dimension_semantics=("parallel",),
        ),
    )(x2, xi, y)
